# Initial kernel scaffold; baseline (speedup 1.0000x reference)
import jax
import jax.numpy as jnp
from jax.experimental import pallas as pl


def kernel(points, batch, pn1_W1, pn1_b1, pn1_W2, pn1_b2, pn1_W3, pn1_b3, pn2_W1, pn2_b1, pn2_W2, pn2_b2, pn2_W3, pn2_b3, pn3_W1, pn3_b1, pn3_W2, pn3_b2, pn3_W3, pn3_b3):
    raise NotImplementedError("write your pallas kernel here")


# speedup vs baseline: 21.9417x; 21.9417x over previous
import jax
import jax.numpy as jnp
from jax.experimental import pallas as pl

_B = 8
_N1 = 2048
_M1 = 512
_M2 = 128
_K = 4


def _mlp(x, params):
    h = x
    for w, b in params:
        h = jnp.maximum(jnp.dot(h, w, preferred_element_type=jnp.float32) + b, 0.0)
    return h


def _fps(px, py, pz, m):
    bc, n = px.shape
    iota_n = jax.lax.broadcasted_iota(jnp.int32, (bc, n), 1)
    iota_m = jax.lax.broadcasted_iota(jnp.int32, (bc, m), 1)

    def gather_cur(last):
        oh = iota_n == last
        cx = jnp.sum(jnp.where(oh, px, 0.0), axis=1, keepdims=True)
        cy = jnp.sum(jnp.where(oh, py, 0.0), axis=1, keepdims=True)
        cz = jnp.sum(jnp.where(oh, pz, 0.0), axis=1, keepdims=True)
        return cx, cy, cz

    def body(i, st):
        dist, last, qx, qy, qz = st
        cx, cy, cz = gather_cur(last)
        put = iota_m == (i - 1)
        qx = jnp.where(put, cx, qx)
        qy = jnp.where(put, cy, qy)
        qz = jnp.where(put, cz, qz)
        dd = (px - cx) ** 2 + (py - cy) ** 2 + (pz - cz) ** 2
        dist = jnp.minimum(dist, dd)
        mx = jnp.max(dist, axis=1, keepdims=True)
        nxt = jnp.min(jnp.where(dist == mx, iota_n, n), axis=1, keepdims=True)
        return dist, nxt, qx, qy, qz

    dist0 = jnp.full((bc, n), jnp.inf, dtype=jnp.float32)
    last0 = jnp.zeros((bc, 1), dtype=jnp.int32)
    q0 = jnp.zeros((bc, m), dtype=jnp.float32)
    _, last, qx, qy, qz = jax.lax.fori_loop(
        1, m, body, (dist0, last0, q0, q0, q0))
    cx, cy, cz = gather_cur(last)
    put = iota_m == (m - 1)
    qx = jnp.where(put, cx, qx)
    qy = jnp.where(put, cy, qy)
    qz = jnp.where(put, cz, qz)
    return qx, qy, qz


def _transpose_planes(qx, qy, qz):
    bc = qx.shape[0]
    r = jax.lax.broadcasted_iota(jnp.int32, (bc, bc), 0)
    c = jax.lax.broadcasted_iota(jnp.int32, (bc, bc), 1)
    eye = jnp.where(r == c, 1.0, 0.0)
    dims = (((0,), (0,)), ((), ()))
    tx = jax.lax.dot_general(qx, eye, dims, preferred_element_type=jnp.float32)
    ty = jax.lax.dot_general(qy, eye, dims, preferred_element_type=jnp.float32)
    tz = jax.lax.dot_general(qz, eye, dims, preferred_element_type=jnp.float32)
    return tx, ty, tz


def _knn_gather_max(qxt, qyt, qzt, b, kx, ky, kz, hb):
    m = qxt.shape[0]
    n = kx.shape[1]
    qx = qxt[:, b:b + 1]
    qy = qyt[:, b:b + 1]
    qz = qzt[:, b:b + 1]
    d = ((qx - kx[b:b + 1, :]) ** 2 + (qy - ky[b:b + 1, :]) ** 2
         + (qz - kz[b:b + 1, :]) ** 2)
    iota = jax.lax.broadcasted_iota(jnp.int32, (m, n), 1)
    acc = None
    for j in range(_K):
        mn = jnp.min(d, axis=1, keepdims=True)
        ij = jnp.min(jnp.where(d == mn, iota, n), axis=1, keepdims=True)
        oh = iota == ij
        g = jnp.dot(jnp.where(oh, 1.0, 0.0), hb,
                    preferred_element_type=jnp.float32)
        acc = g if acc is None else jnp.maximum(acc, g)
        if j < _K - 1:
            d = jnp.where(oh, jnp.inf, d)
    return acc


def _body(pts_ref, px_ref, py_ref, pz_ref, *rest):
    wrefs = rest[:18]
    out_ref = rest[18]
    pts = pts_ref[...]
    px = px_ref[...]
    py = py_ref[...]
    pz = pz_ref[...]
    w = [r[...] for r in wrefs]
    p1 = [(w[0], w[1]), (w[2], w[3]), (w[4], w[5])]
    p2 = [(w[6], w[7]), (w[8], w[9]), (w[10], w[11])]
    p3 = [(w[12], w[13]), (w[14], w[15]), (w[16], w[17])]

    h1 = _mlp(pts, p1)
    q1x, q1y, q1z = _fps(px, py, pz, _M1)
    t1x, t1y, t1z = _transpose_planes(q1x, q1y, q1z)
    f1 = []
    for b in range(_B):
        hb = h1[b * _N1:(b + 1) * _N1, :]
        f1.append(_knn_gather_max(t1x, t1y, t1z, b, px, py, pz, hb))
    f1 = jnp.concatenate(f1, axis=0)

    h2 = _mlp(f1, p2)
    q2x, q2y, q2z = _fps(q1x, q1y, q1z, _M2)
    t2x, t2y, t2z = _transpose_planes(q2x, q2y, q2z)
    f2 = []
    for b in range(_B):
        hb = h2[b * _M1:(b + 1) * _M1, :]
        f2.append(_knn_gather_max(t2x, t2y, t2z, b, q1x, q1y, q1z, hb))
    f2 = jnp.concatenate(f2, axis=0)

    h3 = _mlp(f2, p3)
    for b in range(_B):
        out_ref[b:b + 1, :] = jnp.max(
            h3[b * _M2:(b + 1) * _M2, :], axis=0, keepdims=True)


def kernel(points, batch,
           pn1_W1, pn1_b1, pn1_W2, pn1_b2, pn1_W3, pn1_b3,
           pn2_W1, pn2_b1, pn2_W2, pn2_b2, pn2_W3, pn2_b3,
           pn3_W1, pn3_b1, pn3_W2, pn3_b2, pn3_W3, pn3_b3):
    del batch
    pts3 = points.reshape(_B, _N1, 3)
    px = pts3[:, :, 0]
    py = pts3[:, :, 1]
    pz = pts3[:, :, 2]
    ws = [pn1_W1, pn1_b1.reshape(1, -1), pn1_W2, pn1_b2.reshape(1, -1),
          pn1_W3, pn1_b3.reshape(1, -1),
          pn2_W1, pn2_b1.reshape(1, -1), pn2_W2, pn2_b2.reshape(1, -1),
          pn2_W3, pn2_b3.reshape(1, -1),
          pn3_W1, pn3_b1.reshape(1, -1), pn3_W2, pn3_b2.reshape(1, -1),
          pn3_W3, pn3_b3.reshape(1, -1)]
    return pl.pallas_call(
        _body,
        out_shape=jax.ShapeDtypeStruct((_B, 256), jnp.float32),
    )(points, px, py, pz, *ws)


# speedup vs baseline: 21.9872x; 1.0021x over previous
import jax
import jax.numpy as jnp
from jax.experimental import pallas as pl

_B = 8
_N1 = 2048
_M1 = 512
_M2 = 128
_K = 4


def _mlp(x, params):
    h = x
    for w, b in params:
        h = jnp.maximum(jnp.dot(h, w, preferred_element_type=jnp.float32) + b, 0.0)
    return h


def _fps(px, py, pz, m):
    bc, n = px.shape
    iota_n = jax.lax.broadcasted_iota(jnp.int32, (bc, n), 1)
    iota_m = jax.lax.broadcasted_iota(jnp.int32, (bc, m), 1)

    def gather_cur(last):
        oh = iota_n == last
        cx = jnp.sum(jnp.where(oh, px, 0.0), axis=1, keepdims=True)
        cy = jnp.sum(jnp.where(oh, py, 0.0), axis=1, keepdims=True)
        cz = jnp.sum(jnp.where(oh, pz, 0.0), axis=1, keepdims=True)
        return cx, cy, cz

    def body(i, st):
        dist, last, qx, qy, qz = st
        cx, cy, cz = gather_cur(last)
        put = iota_m == (i - 1)
        qx = jnp.where(put, cx, qx)
        qy = jnp.where(put, cy, qy)
        qz = jnp.where(put, cz, qz)
        dd = (px - cx) ** 2 + (py - cy) ** 2 + (pz - cz) ** 2
        dist = jnp.minimum(dist, dd)
        mx = jnp.max(dist, axis=1, keepdims=True)
        nxt = jnp.min(jnp.where(dist == mx, iota_n, n), axis=1, keepdims=True)
        return dist, nxt, qx, qy, qz

    dist0 = jnp.full((bc, n), jnp.inf, dtype=jnp.float32)
    last0 = jnp.zeros((bc, 1), dtype=jnp.int32)
    q0 = jnp.zeros((bc, m), dtype=jnp.float32)
    _, last, qx, qy, qz = jax.lax.fori_loop(
        1, m, body, (dist0, last0, q0, q0, q0))
    cx, cy, cz = gather_cur(last)
    put = iota_m == (m - 1)
    qx = jnp.where(put, cx, qx)
    qy = jnp.where(put, cy, qy)
    qz = jnp.where(put, cz, qz)
    return qx, qy, qz


def _transpose_planes(qx, qy, qz):
    return jnp.transpose(qx), jnp.transpose(qy), jnp.transpose(qz)


def _knn_gather_max(qxt, qyt, qzt, b, kx, ky, kz, hb):
    m = qxt.shape[0]
    n = kx.shape[1]
    qx = qxt[:, b:b + 1]
    qy = qyt[:, b:b + 1]
    qz = qzt[:, b:b + 1]
    d = ((qx - kx[b:b + 1, :]) ** 2 + (qy - ky[b:b + 1, :]) ** 2
         + (qz - kz[b:b + 1, :]) ** 2)
    iota = jax.lax.broadcasted_iota(jnp.int32, (m, n), 1)
    acc = None
    for j in range(_K):
        mn = jnp.min(d, axis=1, keepdims=True)
        ij = jnp.min(jnp.where(d == mn, iota, n), axis=1, keepdims=True)
        oh = iota == ij
        g = jnp.dot(jnp.where(oh, 1.0, 0.0), hb,
                    preferred_element_type=jnp.float32)
        acc = g if acc is None else jnp.maximum(acc, g)
        if j < _K - 1:
            d = jnp.where(oh, jnp.inf, d)
    return acc


def _body(pts_ref, px_ref, py_ref, pz_ref, *rest):
    wrefs = rest[:18]
    out_ref = rest[18]
    pts = pts_ref[...]
    px = px_ref[...]
    py = py_ref[...]
    pz = pz_ref[...]
    w = [r[...] for r in wrefs]
    p1 = [(w[0], w[1]), (w[2], w[3]), (w[4], w[5])]
    p2 = [(w[6], w[7]), (w[8], w[9]), (w[10], w[11])]
    p3 = [(w[12], w[13]), (w[14], w[15]), (w[16], w[17])]

    h1 = _mlp(pts, p1)
    q1x, q1y, q1z = _fps(px, py, pz, _M1)
    t1x, t1y, t1z = _transpose_planes(q1x, q1y, q1z)
    f1 = []
    for b in range(_B):
        hb = h1[b * _N1:(b + 1) * _N1, :]
        f1.append(_knn_gather_max(t1x, t1y, t1z, b, px, py, pz, hb))
    f1 = jnp.concatenate(f1, axis=0)

    h2 = _mlp(f1, p2)
    q2x, q2y, q2z = _fps(q1x, q1y, q1z, _M2)
    t2x, t2y, t2z = _transpose_planes(q2x, q2y, q2z)
    f2 = []
    for b in range(_B):
        hb = h2[b * _M1:(b + 1) * _M1, :]
        f2.append(_knn_gather_max(t2x, t2y, t2z, b, q1x, q1y, q1z, hb))
    f2 = jnp.concatenate(f2, axis=0)

    h3 = _mlp(f2, p3)
    for b in range(_B):
        out_ref[b:b + 1, :] = jnp.max(
            h3[b * _M2:(b + 1) * _M2, :], axis=0, keepdims=True)


def kernel(points, batch,
           pn1_W1, pn1_b1, pn1_W2, pn1_b2, pn1_W3, pn1_b3,
           pn2_W1, pn2_b1, pn2_W2, pn2_b2, pn2_W3, pn2_b3,
           pn3_W1, pn3_b1, pn3_W2, pn3_b2, pn3_W3, pn3_b3):
    del batch
    pts3 = points.reshape(_B, _N1, 3)
    px = pts3[:, :, 0]
    py = pts3[:, :, 1]
    pz = pts3[:, :, 2]
    ws = [pn1_W1, pn1_b1.reshape(1, -1), pn1_W2, pn1_b2.reshape(1, -1),
          pn1_W3, pn1_b3.reshape(1, -1),
          pn2_W1, pn2_b1.reshape(1, -1), pn2_W2, pn2_b2.reshape(1, -1),
          pn2_W3, pn2_b3.reshape(1, -1),
          pn3_W1, pn3_b1.reshape(1, -1), pn3_W2, pn3_b2.reshape(1, -1),
          pn3_W3, pn3_b3.reshape(1, -1)]
    return pl.pallas_call(
        _body,
        out_shape=jax.ShapeDtypeStruct((_B, 256), jnp.float32),
    )(points, px, py, pz, *ws)
